# baseline (speedup 1.0000x reference)
import functools

import jax
import jax.numpy as jnp
from jax import lax
from jax.experimental import pallas as pl
from jax.experimental.pallas import tpu as pltpu
from jax.experimental.pallas import tpu_sc as plsc

N = 10000
E = 320000
H = 128

NP = 10240
EP = 327680

NC = 2
NS = 16
NW = NC * NS
CH = 128
EPW = EP // NW
NCHUNK = EPW // CH
NACC = 10112
RPS = NACC // NS

RB = 1024


def _dot(a, b):
    return jnp.dot(a, b, preferred_element_type=jnp.float32)


def _tc1_body(x_ref, wa_ref, ba_ref, wb_ref, bb_ref, o_ref):
    h = jnp.maximum(_dot(x_ref[...], wa_ref[...]) + ba_ref[...], 0.0)
    o_ref[...] = _dot(h, wb_ref[...]) + bb_ref[...]


def _tc2_body(x_ref, a0_ref, a1_ref, ux_ref, ua_ref, ub_ref,
              wa_ref, ba_ref, wb_ref, bb_ref, h1_ref, m2_ref):
    a = a0_ref[0] + a1_ref[0]
    h1 = jnp.maximum(_dot(x_ref[...], ux_ref[...]) + _dot(a, ua_ref[...])
                     + ub_ref[...], 0.0)
    h1_ref[...] = h1
    h = jnp.maximum(_dot(h1, wa_ref[...]) + ba_ref[...], 0.0)
    m2_ref[...] = _dot(h, wb_ref[...]) + bb_ref[...]


def _tc3_body(h1_ref, a0_ref, a1_ref, ux_ref, ua_ref, ub_ref,
              wo_ref, bo_ref, o_ref):
    a = a0_ref[0] + a1_ref[0]
    h2 = jnp.maximum(_dot(h1_ref[...], ux_ref[...]) + _dot(a, ua_ref[...])
                     + ub_ref[...], 0.0)
    o_ref[...] = _dot(h2, wo_ref[...]) + bo_ref[...]


def _full(shape):
    return pl.BlockSpec(shape, lambda i: (0,) * len(shape))


def _tc1(x_pad, wa, ba, wb, bb):
    return pl.pallas_call(
        _tc1_body,
        grid=(NP // RB,),
        in_specs=[
            pl.BlockSpec((RB, 256), lambda i: (i, 0)),
            _full((256, H)), _full((1, H)), _full((H, H)), _full((1, H)),
        ],
        out_specs=pl.BlockSpec((RB, H), lambda i: (i, 0)),
        out_shape=jax.ShapeDtypeStruct((NP, H), jnp.float32),
    )(x_pad, wa, ba, wb, bb)


def _tc2(x_pad, aggr, ux, ua, ub, wa, ba, wb, bb):
    return pl.pallas_call(
        _tc2_body,
        grid=(NP // RB,),
        in_specs=[
            pl.BlockSpec((RB, 256), lambda i: (i, 0)),
            pl.BlockSpec((1, RB, H), lambda i: (0, i, 0)),
            pl.BlockSpec((1, RB, H), lambda i: (1, i, 0)),
            _full((256, H)), _full((H, H)), _full((1, H)),
            _full((H, H)), _full((1, H)), _full((H, H)), _full((1, H)),
        ],
        out_specs=[
            pl.BlockSpec((RB, H), lambda i: (i, 0)),
            pl.BlockSpec((RB, H), lambda i: (i, 0)),
        ],
        out_shape=[
            jax.ShapeDtypeStruct((NP, H), jnp.float32),
            jax.ShapeDtypeStruct((NP, H), jnp.float32),
        ],
    )(x_pad, aggr, aggr, ux, ua, ub, wa, ba, wb, bb)


def _tc3(h1, aggr, ux, ua, ub, wo, bo):
    return pl.pallas_call(
        _tc3_body,
        grid=(NP // RB,),
        in_specs=[
            pl.BlockSpec((RB, H), lambda i: (i, 0)),
            pl.BlockSpec((1, RB, H), lambda i: (0, i, 0)),
            pl.BlockSpec((1, RB, H), lambda i: (1, i, 0)),
            _full((H, H)), _full((H, H)), _full((1, H)),
            _full((H, H)), _full((1, H)),
        ],
        out_specs=pl.BlockSpec((RB, H), lambda i: (i, 0)),
        out_shape=jax.ShapeDtypeStruct((NP, H), jnp.float32),
    )(h1, aggr, aggr, ux, ua, ub, wo, bo)


def _sc_aggregate(msg, src, dst):
    mesh = plsc.VectorSubcoreMesh(core_axis_name="c", subcore_axis_name="s",
                                  num_cores=NC, num_subcores=NS)

    @functools.partial(
        pl.kernel,
        out_type=jax.ShapeDtypeStruct((NC, NP, H), jnp.float32),
        mesh=mesh,
        scratch_types=[
            [pltpu.VMEM((CH,), jnp.int32)] * 3,
            [pltpu.VMEM((CH,), jnp.int32)] * 3,
            [pltpu.VMEM((CH, H), jnp.float32)] * 3,
            pltpu.VMEM_SHARED((NACC, H), jnp.float32),
            [pltpu.SemaphoreType.DMA] * 3,
            [pltpu.SemaphoreType.DMA] * 3,
            [pltpu.SemaphoreType.DMA] * 3,
        ],
    )
    def k(msg_hbm, src_hbm, dst_hbm, out_hbm, sidx, didx, rows, acc, sem,
          isem, dsem):
        cid = lax.axis_index("c")
        sid = lax.axis_index("s")
        wid = cid * NS + sid

        @pl.loop(0, CH)
        def _(i):
            @pl.loop(0, H, step=16)
            def _(j):
                rows[0][i, pl.ds(j, 16)] = jnp.zeros((16,), jnp.float32)

        rbase = sid * RPS
        for r in range(RPS // CH):
            pltpu.sync_copy(rows[0], acc.at[pl.ds(rbase + r * CH, CH)])
        rem = RPS % CH
        if rem:
            pltpu.sync_copy(rows[0].at[pl.ds(0, rem)],
                            acc.at[pl.ds(rbase + (RPS // CH) * CH, rem)])
        plsc.subcore_barrier()

        cbase = wid * NCHUNK

        def pidx(ci, b):
            pltpu.async_copy(src_hbm.at[ci], sidx[b], isem[b])
            pltpu.async_copy(dst_hbm.at[ci], didx[b], dsem[b])

        def gat(ci, b):
            pltpu.make_async_copy(src_hbm.at[ci], sidx[b], isem[b]).wait()
            pltpu.async_copy(msg_hbm.at[sidx[b]], rows[b], sem[b])

        def fin(ci, b):
            pltpu.make_async_copy(msg_hbm.at[sidx[b]], rows[b], sem[b]).wait()
            pltpu.make_async_copy(dst_hbm.at[ci], didx[b], dsem[b]).wait()
            pltpu.sync_copy(rows[b], acc.at[didx[b]], add=True)

        pidx(cbase, 0)
        pidx(cbase + 1, 1)
        gat(cbase, 0)

        @pl.loop(0, NCHUNK - 2, step=3)
        def _(ci):
            pidx(cbase + ci + 2, 2)
            gat(cbase + ci + 1, 1)
            fin(cbase + ci, 0)
            pidx(cbase + ci + 3, 0)
            gat(cbase + ci + 2, 2)
            fin(cbase + ci + 1, 1)
            pidx(cbase + ci + 4, 1)
            gat(cbase + ci + 3, 0)
            fin(cbase + ci + 2, 2)

        gat(cbase + NCHUNK - 1, 1)
        fin(cbase + NCHUNK - 2, 0)
        fin(cbase + NCHUNK - 1, 1)

        plsc.subcore_barrier()
        pltpu.sync_copy(acc.at[pl.ds(rbase, RPS)],
                        out_hbm.at[cid, pl.ds(rbase, RPS)])

    return k(msg, src, dst)


def kernel(x, edge_index, noise, W1a, b1a, W1b, b1b, U1, u1b,
           W2a, b2a, W2b, b2b, U2, u2b, Wo, bo):
    f32 = jnp.float32
    in1 = x.shape[1] + noise.shape[1]

    xw = jnp.concatenate([x, noise], axis=1)
    x_pad = jnp.zeros((NP, 256), f32).at[:N, :in1].set(xw)

    pad = jnp.arange(EP - E, dtype=jnp.int32)
    src = jnp.concatenate([edge_index[0], pad % N])
    dst = jnp.concatenate([edge_index[1], N + pad % (NACC - N)])
    src = src.reshape(NW * NCHUNK, CH)
    dst = dst.reshape(NW * NCHUNK, CH)

    w1a = jnp.zeros((256, H), f32).at[:in1].set(W1a)
    u1x = jnp.zeros((256, H), f32).at[:in1].set(U1[:in1])
    u1a = U1[in1:]
    u2x = U2[:H]
    u2a = U2[H:]
    wo = jnp.zeros((H, H), f32).at[:, :1].set(Wo)
    bo_pad = jnp.zeros((1, H), f32).at[0, 0].set(bo[0])
    r = lambda b: b.reshape(1, H)

    msg1 = _tc1(x_pad, w1a, r(b1a), W1b, r(b1b))
    aggr1 = _sc_aggregate(msg1, src, dst)
    h1, msg2 = _tc2(x_pad, aggr1, u1x, u1a, r(u1b), W2a, r(b2a), W2b, r(b2b))
    aggr2 = _sc_aggregate(msg2, src, dst)
    out = _tc3(h1, aggr2, u2x, u2a, r(u2b), wo, bo_pad)
    return out[:N, :1]


# speedup vs baseline: 1.0520x; 1.0520x over previous
import functools

import jax
import jax.numpy as jnp
from jax import lax
from jax.experimental import pallas as pl
from jax.experimental.pallas import tpu as pltpu
from jax.experimental.pallas import tpu_sc as plsc

N = 10000
E = 320000
H = 128

NP = 10240
EP = 327680

NC = 2
NS = 16
NW = NC * NS
CH = 128
EPW = EP // NW
NCHUNK = EPW // CH
NACC = 10112
RPS = NACC // NS

RB = 2048


def _dot(a, b):
    return jnp.dot(a, b, preferred_element_type=jnp.float32)


def _tc1_body(x_ref, wa_ref, ba_ref, wb_ref, bb_ref, o_ref):
    h = jnp.maximum(_dot(x_ref[...], wa_ref[...]) + ba_ref[...], 0.0)
    o_ref[...] = _dot(h, wb_ref[...]) + bb_ref[...]


def _tc2_body(x_ref, a0_ref, a1_ref, ux_ref, ua_ref, ub_ref,
              wa_ref, ba_ref, wb_ref, bb_ref, h1_ref, m2_ref):
    a = a0_ref[0] + a1_ref[0]
    h1 = jnp.maximum(_dot(x_ref[...], ux_ref[...]) + _dot(a, ua_ref[...])
                     + ub_ref[...], 0.0)
    h1_ref[...] = h1
    h = jnp.maximum(_dot(h1, wa_ref[...]) + ba_ref[...], 0.0)
    m2_ref[...] = _dot(h, wb_ref[...]) + bb_ref[...]


def _tc3_body(h1_ref, a0_ref, a1_ref, ux_ref, ua_ref, ub_ref,
              wo_ref, bo_ref, o_ref):
    a = a0_ref[0] + a1_ref[0]
    h2 = jnp.maximum(_dot(h1_ref[...], ux_ref[...]) + _dot(a, ua_ref[...])
                     + ub_ref[...], 0.0)
    o_ref[...] = _dot(h2, wo_ref[...]) + bo_ref[...]


def _full(shape):
    return pl.BlockSpec(shape, lambda i: (0,) * len(shape))


def _tc1(x_pad, wa, ba, wb, bb):
    return pl.pallas_call(
        _tc1_body,
        grid=(NP // RB,),
        in_specs=[
            pl.BlockSpec((RB, 256), lambda i: (i, 0)),
            _full((256, H)), _full((1, H)), _full((H, H)), _full((1, H)),
        ],
        out_specs=pl.BlockSpec((RB, H), lambda i: (i, 0)),
        out_shape=jax.ShapeDtypeStruct((NP, H), jnp.float32),
    )(x_pad, wa, ba, wb, bb)


def _tc2(x_pad, aggr, ux, ua, ub, wa, ba, wb, bb):
    return pl.pallas_call(
        _tc2_body,
        grid=(NP // RB,),
        in_specs=[
            pl.BlockSpec((RB, 256), lambda i: (i, 0)),
            pl.BlockSpec((1, RB, H), lambda i: (0, i, 0)),
            pl.BlockSpec((1, RB, H), lambda i: (1, i, 0)),
            _full((256, H)), _full((H, H)), _full((1, H)),
            _full((H, H)), _full((1, H)), _full((H, H)), _full((1, H)),
        ],
        out_specs=[
            pl.BlockSpec((RB, H), lambda i: (i, 0)),
            pl.BlockSpec((RB, H), lambda i: (i, 0)),
        ],
        out_shape=[
            jax.ShapeDtypeStruct((NP, H), jnp.float32),
            jax.ShapeDtypeStruct((NP, H), jnp.float32),
        ],
    )(x_pad, aggr, aggr, ux, ua, ub, wa, ba, wb, bb)


def _tc3(h1, aggr, ux, ua, ub, wo, bo):
    return pl.pallas_call(
        _tc3_body,
        grid=(NP // RB,),
        in_specs=[
            pl.BlockSpec((RB, H), lambda i: (i, 0)),
            pl.BlockSpec((1, RB, H), lambda i: (0, i, 0)),
            pl.BlockSpec((1, RB, H), lambda i: (1, i, 0)),
            _full((H, H)), _full((H, H)), _full((1, H)),
            _full((H, H)), _full((1, H)),
        ],
        out_specs=pl.BlockSpec((RB, H), lambda i: (i, 0)),
        out_shape=jax.ShapeDtypeStruct((NP, H), jnp.float32),
    )(h1, aggr, aggr, ux, ua, ub, wo, bo)


def _sc_aggregate(msg, src, dst):
    mesh = plsc.VectorSubcoreMesh(core_axis_name="c", subcore_axis_name="s",
                                  num_cores=NC, num_subcores=NS)

    @functools.partial(
        pl.kernel,
        out_type=jax.ShapeDtypeStruct((NC, NP, H), jnp.float32),
        mesh=mesh,
        scratch_types=[
            [pltpu.VMEM((CH,), jnp.int32)] * 3,
            [pltpu.VMEM((CH,), jnp.int32)] * 3,
            [pltpu.VMEM((CH, H), jnp.float32)] * 3,
            pltpu.VMEM_SHARED((NACC, H), jnp.float32),
            [pltpu.SemaphoreType.DMA] * 3,
            [pltpu.SemaphoreType.DMA] * 3,
            [pltpu.SemaphoreType.DMA] * 3,
            [pltpu.SemaphoreType.DMA] * 3,
        ],
    )
    def k(msg_hbm, src_hbm, dst_hbm, out_hbm, sidx, didx, rows, acc, sem,
          isem, dsem, ssem):
        cid = lax.axis_index("c")
        sid = lax.axis_index("s")
        wid = cid * NS + sid

        @pl.loop(0, CH)
        def _(i):
            @pl.loop(0, H, step=16)
            def _(j):
                rows[0][i, pl.ds(j, 16)] = jnp.zeros((16,), jnp.float32)

        rbase = sid * RPS
        for r in range(RPS // CH):
            pltpu.sync_copy(rows[0], acc.at[pl.ds(rbase + r * CH, CH)])
        rem = RPS % CH
        if rem:
            pltpu.sync_copy(rows[0].at[pl.ds(0, rem)],
                            acc.at[pl.ds(rbase + (RPS // CH) * CH, rem)])
        plsc.subcore_barrier()

        cbase = wid * NCHUNK

        def pidx(ci, b):
            pltpu.async_copy(src_hbm.at[ci], sidx[b], isem[b])
            pltpu.async_copy(dst_hbm.at[ci], didx[b], dsem[b])

        def gat(ci, b, drain=True):
            if drain:
                pltpu.make_async_copy(rows[b], acc.at[didx[b]],
                                      ssem[b]).wait()
            pltpu.make_async_copy(src_hbm.at[ci], sidx[b], isem[b]).wait()
            pltpu.async_copy(msg_hbm.at[sidx[b]], rows[b], sem[b])

        def fin(ci, b):
            pltpu.make_async_copy(msg_hbm.at[sidx[b]], rows[b], sem[b]).wait()
            pltpu.make_async_copy(dst_hbm.at[ci], didx[b], dsem[b]).wait()
            pltpu.async_copy(rows[b], acc.at[didx[b]], ssem[b], add=True)

        pidx(cbase, 0)
        pidx(cbase + 1, 1)
        gat(cbase, 0, drain=False)
        pidx(cbase + 2, 2)
        gat(cbase + 1, 1, drain=False)
        fin(cbase, 0)
        pidx(cbase + 3, 0)
        gat(cbase + 2, 2, drain=False)
        fin(cbase + 1, 1)
        pidx(cbase + 4, 1)
        gat(cbase + 3, 0)
        fin(cbase + 2, 2)

        @pl.loop(3, NCHUNK - 2, step=3)
        def _(ci):
            pidx(cbase + ci + 2, 2)
            gat(cbase + ci + 1, 1)
            fin(cbase + ci, 0)
            pidx(cbase + ci + 3, 0)
            gat(cbase + ci + 2, 2)
            fin(cbase + ci + 1, 1)
            pidx(cbase + ci + 4, 1)
            gat(cbase + ci + 3, 0)
            fin(cbase + ci + 2, 2)

        gat(cbase + NCHUNK - 1, 1)
        fin(cbase + NCHUNK - 2, 0)
        fin(cbase + NCHUNK - 1, 1)
        pltpu.make_async_copy(rows[0], acc.at[didx[0]], ssem[0]).wait()
        pltpu.make_async_copy(rows[1], acc.at[didx[1]], ssem[1]).wait()
        pltpu.make_async_copy(rows[2], acc.at[didx[2]], ssem[2]).wait()

        plsc.subcore_barrier()
        pltpu.sync_copy(acc.at[pl.ds(rbase, RPS)],
                        out_hbm.at[cid, pl.ds(rbase, RPS)])

    return k(msg, src, dst)


def kernel(x, edge_index, noise, W1a, b1a, W1b, b1b, U1, u1b,
           W2a, b2a, W2b, b2b, U2, u2b, Wo, bo):
    f32 = jnp.float32
    in1 = x.shape[1] + noise.shape[1]

    xw = jnp.concatenate([x, noise], axis=1)
    x_pad = jnp.zeros((NP, 256), f32).at[:N, :in1].set(xw)

    pad = jnp.arange(EP - E, dtype=jnp.int32)
    src = jnp.concatenate([edge_index[0], pad % N])
    dst = jnp.concatenate([edge_index[1], N + pad % (NACC - N)])
    src = src.reshape(NW * NCHUNK, CH)
    dst = dst.reshape(NW * NCHUNK, CH)

    w1a = jnp.zeros((256, H), f32).at[:in1].set(W1a)
    u1x = jnp.zeros((256, H), f32).at[:in1].set(U1[:in1])
    u1a = U1[in1:]
    u2x = U2[:H]
    u2a = U2[H:]
    wo = jnp.zeros((H, H), f32).at[:, :1].set(Wo)
    bo_pad = jnp.zeros((1, H), f32).at[0, 0].set(bo[0])
    r = lambda b: b.reshape(1, H)

    msg1 = _tc1(x_pad, w1a, r(b1a), W1b, r(b1b))
    aggr1 = _sc_aggregate(msg1, src, dst)
    h1, msg2 = _tc2(x_pad, aggr1, u1x, u1a, r(u1b), W2a, r(b2a), W2b, r(b2b))
    aggr2 = _sc_aggregate(msg2, src, dst)
    out = _tc3(h1, aggr2, u2x, u2a, r(u2b), wo, bo_pad)
    return out[:N, :1]
